# baseline (speedup 1.0000x reference)
import jax
import jax.numpy as jnp
from jax import lax
from jax.experimental import pallas as pl
from jax.experimental.pallas import tpu as pltpu

_NUM_BINS = 64
_N = 1048576
_C = 8192
_R = _N // _C
_BR = 16
_STEPS = _R // _BR


def _tc_body(x_ref, w_ref):
    x = x_ref[...]
    t = x * jnp.float32(_NUM_BINS)
    fi = t.astype(jnp.int32)
    exact = fi.astype(jnp.float32) == t
    b = jnp.maximum(jnp.where(exact, fi - 1, fi), 0)
    one = jnp.int32(1)
    lo = jnp.where(b < 32, one << jnp.minimum(b, 31), 0)
    hi = jnp.where(b >= 32, one << jnp.maximum(b - 32, 0), 0)
    w_ref[0, :, :] = lo
    w_ref[1, :, :] = hi


def kernel(sm_vector):
    x2d = sm_vector.reshape(_R, _C)
    w = pl.pallas_call(
        _tc_body,
        grid=(_STEPS,),
        in_specs=[pl.BlockSpec((_BR, _C), lambda j: (j, 0))],
        out_specs=pl.BlockSpec((2, _BR, _C), lambda j: (0, j, 0)),
        out_shape=jax.ShapeDtypeStruct((2, _R, _C), jnp.int32),
        compiler_params=pltpu.CompilerParams(
            dimension_semantics=("arbitrary",),
        ),
    )(x2d)
    w2 = w.reshape(2, _N)
    bit = (jnp.int32(1) << jnp.arange(32, dtype=jnp.int32))[:, None]
    lo_m = (w2[0][None, :] & bit) != 0
    hi_m = (w2[1][None, :] & bit) != 0
    return jnp.concatenate([lo_m, hi_m], axis=0)


# speedup vs baseline: 1.5449x; 1.5449x over previous
import jax
import jax.numpy as jnp
from jax import lax
from jax.experimental import pallas as pl
from jax.experimental.pallas import tpu as pltpu

_NUM_BINS = 64
_N = 1048576
_C = 8192
_R = _N // _C
_BR = 16
_STEPS = _R // _BR


def _tc_body(x_ref, wl_ref, wh_ref):
    x = x_ref[...]
    t = x * jnp.float32(_NUM_BINS)
    fi = t.astype(jnp.int32)
    exact = fi.astype(jnp.float32) == t
    b = jnp.maximum(jnp.where(exact, fi - 1, fi), 0)
    one = jnp.int32(1)
    wl_ref[...] = jnp.where(b < 32, one << jnp.minimum(b, 31), 0)
    wh_ref[...] = jnp.where(b >= 32, one << jnp.maximum(b - 32, 0), 0)


def kernel(sm_vector):
    x2d = sm_vector.reshape(_R, _C)
    wl, wh = pl.pallas_call(
        _tc_body,
        grid=(_STEPS,),
        in_specs=[pl.BlockSpec((_BR, _C), lambda j: (j, 0))],
        out_specs=[
            pl.BlockSpec((_BR, _C), lambda j: (j, 0)),
            pl.BlockSpec((_BR, _C), lambda j: (j, 0)),
        ],
        out_shape=[
            jax.ShapeDtypeStruct((_R, _C), jnp.int32),
            jax.ShapeDtypeStruct((_R, _C), jnp.int32),
        ],
        compiler_params=pltpu.CompilerParams(
            dimension_semantics=("arbitrary",),
        ),
    )(x2d)
    wlf = wl.reshape(_N)
    whf = wh.reshape(_N)
    i = jnp.arange(_NUM_BINS, dtype=jnp.int32)
    one = jnp.int32(1)
    m_lo = jnp.where(i < 32, one << jnp.minimum(i, 31), 0)[:, None]
    m_hi = jnp.where(i >= 32, one << jnp.maximum(i - 32, 0), 0)[:, None]
    return ((wlf[None, :] & m_lo) | (whf[None, :] & m_hi)) != 0


# speedup vs baseline: 1.7132x; 1.1089x over previous
import jax
import jax.numpy as jnp
from jax import lax
from jax.experimental import pallas as pl
from jax.experimental.pallas import tpu as pltpu

_NUM_BINS = 64
_N = 1048576
_C = 8192
_R = _N // _C
_BR = 16
_STEPS = _R // _BR


def _tc_body(x_ref, wl_ref, wh_ref):
    x = x_ref[...]
    t = x * jnp.float32(_NUM_BINS)
    fi = t.astype(jnp.int32)
    exact = fi.astype(jnp.float32) == t
    b = jnp.maximum(jnp.where(exact, fi - 1, fi), 0)
    one = jnp.int32(1)
    wl_ref[...] = jnp.where(b < 32, one << jnp.minimum(b, 31), 0)
    wh_ref[...] = jnp.where(b >= 32, one << jnp.maximum(b - 32, 0), 0)


def kernel(sm_vector):
    x2d = sm_vector.reshape(_R, _C)
    wl, wh = pl.pallas_call(
        _tc_body,
        grid=(_STEPS,),
        in_specs=[pl.BlockSpec((_BR, _C), lambda j: (j, 0))],
        out_specs=[
            pl.BlockSpec((_BR, _C), lambda j: (j, 0)),
            pl.BlockSpec((_BR, _C), lambda j: (j, 0)),
        ],
        out_shape=[
            jax.ShapeDtypeStruct((_R, _C), jnp.int32),
            jax.ShapeDtypeStruct((_R, _C), jnp.int32),
        ],
        compiler_params=pltpu.CompilerParams(
            dimension_semantics=("arbitrary",),
        ),
    )(x2d)
    wlf = wl.reshape(_N)
    whf = wh.reshape(_N)
    i = jnp.arange(_NUM_BINS, dtype=jnp.int32)
    one = jnp.int32(1)
    bit = (one << (i & 31))[:, None]
    picked = jnp.where((i < 32)[:, None], wlf[None, :], whf[None, :])
    return (picked & bit) != 0
